# Initial kernel scaffold; baseline (speedup 1.0000x reference)
import jax
import jax.numpy as jnp
from jax.experimental import pallas as pl


def kernel(node_features, edges_power, edges_comm, gnn_Wself_0, gnn_Wrel_0_power, gnn_Wfilm_0_power, gnn_Wrel_0_comm, gnn_Wfilm_0_comm, gnn_Wself_1, gnn_Wrel_1_power, gnn_Wfilm_1_power, gnn_Wrel_1_comm, gnn_Wfilm_1_comm, mlp_W0, mlp_b0, mlp_W1, mlp_b1, mlp_W2, mlp_b2, mlp_W3, mlp_b3, mlp_W4, mlp_b4, mlp_W5, mlp_b5):
    raise NotImplementedError("write your pallas kernel here")


# speedup vs baseline: 10.0409x; 10.0409x over previous
import functools

import jax
import jax.numpy as jnp
from jax import lax
from jax.experimental import pallas as pl
from jax.experimental.pallas import tpu as pltpu
from jax.experimental.pallas import tpu_sc as plsc

N = 50000
E = 1600000
HID = 32
NC = 2
NS = 16
NW = NC * NS
EPT = E // NW
CH = 128
NFULL = EPT // CH
TAIL = EPT - NFULL * CH
ROWS0 = 3128
ROWS0_LAST = N - 15 * ROWS0
NP1 = 51200
RPT1 = NP1 // NS
RB = 2000
GRID = N // RB

_mesh = plsc.VectorSubcoreMesh(core_axis_name="c", subcore_axis_name="s")


def _prep0_body(nf, wself, wrp, wfp, wrc, wfc, s0, ap, gbp, ac, gbc):
    h = nf[...]
    dot = lambda w: jnp.dot(h, w[...], preferred_element_type=jnp.float32)
    s0[...] = dot(wself)
    ap[...] = dot(wrp)
    gbp[...] = dot(wfp)
    ac[...] = dot(wrc)
    gbc[...] = dot(wfc)


def _prep0(nf, wself, wrp, wfp, wrc, wfc):
    full = lambda shape: pl.BlockSpec(shape, lambda k: (0, 0))
    return pl.pallas_call(
        _prep0_body,
        grid=(GRID,),
        in_specs=[
            pl.BlockSpec((RB, 5), lambda k: (k, 0)),
            full((5, HID)), full((5, HID)), full((5, 2 * HID)),
            full((5, HID)), full((5, 2 * HID)),
        ],
        out_specs=[
            pl.BlockSpec((RB, HID), lambda k: (k, 0)),
            pl.BlockSpec((RB, HID), lambda k: (k, 0)),
            pl.BlockSpec((RB, 2 * HID), lambda k: (k, 0)),
            pl.BlockSpec((RB, HID), lambda k: (k, 0)),
            pl.BlockSpec((RB, 2 * HID), lambda k: (k, 0)),
        ],
        out_shape=[
            jax.ShapeDtypeStruct((N, HID), jnp.float32),
            jax.ShapeDtypeStruct((N, HID), jnp.float32),
            jax.ShapeDtypeStruct((N, 2 * HID), jnp.float32),
            jax.ShapeDtypeStruct((N, HID), jnp.float32),
            jax.ShapeDtypeStruct((N, 2 * HID), jnp.float32),
        ],
    )(nf, wself, wrp, wfp, wrc, wfc)


@functools.partial(
    pl.kernel,
    out_type=jax.ShapeDtypeStruct((NC, N, HID), jnp.float32),
    mesh=_mesh,
    scratch_types=[
        pltpu.VMEM((CH,), jnp.int32),
        pltpu.VMEM((CH,), jnp.int32),
        pltpu.VMEM((CH, HID), jnp.float32),
        pltpu.VMEM((CH, 2 * HID), jnp.float32),
        pltpu.VMEM((CH, HID), jnp.float32),
        pltpu.VMEM((TAIL,), jnp.int32),
        pltpu.VMEM((TAIL,), jnp.int32),
        pltpu.VMEM((TAIL, HID), jnp.float32),
        pltpu.VMEM((TAIL, 2 * HID), jnp.float32),
        pltpu.VMEM((TAIL, HID), jnp.float32),
        pltpu.VMEM_SHARED((N, HID), jnp.float32),
        pltpu.SemaphoreType.DMA,
    ],
    compiler_params=pltpu.CompilerParams(use_tc_tiling_on_sc=False),
)
def _edges0(srcp, dstp, srcc, dstc, a_p, gb_p, a_c, gb_c, z0, out,
            sv, dv, av, gbv, mv, svt, dvt, avt, gbvt, mvt, acc, sem):
    c = lax.axis_index("c")
    s = lax.axis_index("s")
    wid = c * NS + s

    @pl.when(s < NS - 1)
    def _():
        pltpu.sync_copy(z0.at[pl.ds(s * ROWS0, ROWS0)],
                        acc.at[pl.ds(s * ROWS0, ROWS0)])

    @pl.when(s == NS - 1)
    def _():
        pltpu.sync_copy(z0.at[pl.ds((NS - 1) * ROWS0, ROWS0_LAST)],
                        acc.at[pl.ds((NS - 1) * ROWS0, ROWS0_LAST)])

    plsc.subcore_barrier()

    def run_chunk(src_h, dst_h, a_h, gb_h, off, k, svk, dvk, avk, gbvk, mvk):
        pltpu.sync_copy(src_h.at[pl.ds(off, k)], svk)
        pltpu.sync_copy(dst_h.at[pl.ds(off, k)], dvk)
        cp1 = pltpu.async_copy(a_h.at[svk], avk, sem)
        cp2 = pltpu.async_copy(gb_h.at[dvk], gbvk, sem)
        cp1.wait()
        cp2.wait()

        def body(i, carry):
            a0 = avk[i, pl.ds(0, 16)]
            a1 = avk[i, pl.ds(16, 16)]
            g0 = gbvk[i, pl.ds(0, 16)]
            g1 = gbvk[i, pl.ds(16, 16)]
            b0 = gbvk[i, pl.ds(32, 16)]
            b1 = gbvk[i, pl.ds(48, 16)]
            mvk[i, pl.ds(0, 16)] = jnp.maximum(g0 * a0 + b0, 0.0)
            mvk[i, pl.ds(16, 16)] = jnp.maximum(g1 * a1 + b1, 0.0)
            return carry

        lax.fori_loop(0, k, body, 0)
        pltpu.sync_copy(mvk, acc.at[dvk], add=True)

    for src_h, dst_h, a_h, gb_h in ((srcp, dstp, a_p, gb_p),
                                    (srcc, dstc, a_c, gb_c)):
        base = wid * EPT

        def chunk_loop(j, carry):
            run_chunk(src_h, dst_h, a_h, gb_h, base + j * CH, CH,
                      sv, dv, av, gbv, mv)
            return carry

        lax.fori_loop(0, NFULL, chunk_loop, 0)
        run_chunk(src_h, dst_h, a_h, gb_h, base + NFULL * CH, TAIL,
                  svt, dvt, avt, gbvt, mvt)

    plsc.subcore_barrier()

    @pl.when(s < NS - 1)
    def _():
        pltpu.sync_copy(acc.at[pl.ds(s * ROWS0, ROWS0)],
                        out.at[c, pl.ds(s * ROWS0, ROWS0)])

    @pl.when(s == NS - 1)
    def _():
        pltpu.sync_copy(acc.at[pl.ds((NS - 1) * ROWS0, ROWS0_LAST)],
                        out.at[c, pl.ds((NS - 1) * ROWS0, ROWS0_LAST)])


def _prep1_body(s0, parts, wt, t_out, s1_out):
    h1 = jnp.maximum(s0[...] + parts[0] + parts[1], 0.0)
    t = jnp.dot(h1, wt[...], preferred_element_type=jnp.float32)
    t_out[...] = t
    s1_out[...] = t[:, 0:1]


def _prep1(s0, parts, wt):
    return pl.pallas_call(
        _prep1_body,
        grid=(GRID,),
        in_specs=[
            pl.BlockSpec((RB, HID), lambda k: (k, 0)),
            pl.BlockSpec((NC, RB, HID), lambda k: (0, k, 0)),
            pl.BlockSpec((HID, 16), lambda k: (0, 0)),
        ],
        out_specs=[
            pl.BlockSpec((RB, 16), lambda k: (k, 0)),
            pl.BlockSpec((RB, 1), lambda k: (k, 0)),
        ],
        out_shape=[
            jax.ShapeDtypeStruct((N, 16), jnp.float32),
            jax.ShapeDtypeStruct((N, 1), jnp.float32),
        ],
    )(s0, parts, wt)


@functools.partial(
    pl.kernel,
    out_type=jax.ShapeDtypeStruct((NC, NP1), jnp.float32),
    mesh=_mesh,
    scratch_types=[
        pltpu.VMEM((CH,), jnp.int32),
        pltpu.VMEM((CH,), jnp.int32),
        pltpu.VMEM((CH, 16), jnp.float32),
        pltpu.VMEM((CH, 16), jnp.float32),
        pltpu.VMEM((CH,), jnp.float32),
        pltpu.VMEM((TAIL,), jnp.int32),
        pltpu.VMEM((TAIL,), jnp.int32),
        pltpu.VMEM((TAIL, 16), jnp.float32),
        pltpu.VMEM((TAIL, 16), jnp.float32),
        pltpu.VMEM((TAIL,), jnp.float32),
        pltpu.VMEM_SHARED((NP1,), jnp.float32),
        pltpu.SemaphoreType.DMA,
    ],
    compiler_params=pltpu.CompilerParams(use_tc_tiling_on_sc=False,
                                         needs_layout_passes=False),
)
def _edges1(srcp, dstp, srcc, dstc, tbl, z1, out,
            sv, dv, tsv, tdv, mv, svt, dvt, tsvt, tdvt, mvt, acc, sem):
    c = lax.axis_index("c")
    s = lax.axis_index("s")
    wid = c * NS + s
    pltpu.sync_copy(z1.at[pl.ds(s * RPT1, RPT1)],
                    acc.at[pl.ds(s * RPT1, RPT1)])
    plsc.subcore_barrier()

    def run_chunk(src_h, dst_h, acol, gcol, bcol, off, k,
                  svk, dvk, tsk, tdk, mvk):
        pltpu.sync_copy(src_h.at[pl.ds(off, k)], svk)
        pltpu.sync_copy(dst_h.at[pl.ds(off, k)], dvk)
        cp1 = pltpu.async_copy(tbl.at[svk], tsk, sem)
        cp2 = pltpu.async_copy(tbl.at[dvk], tdk, sem)
        cp1.wait()
        cp2.wait()
        iot = lax.iota(jnp.int32, 16)
        for j in range(k // 16):
            rows = j * 16 + iot
            a = plsc.load_gather(tsk, [rows, jnp.full((16,), acol, jnp.int32)])
            g = plsc.load_gather(tdk, [rows, jnp.full((16,), gcol, jnp.int32)])
            b = plsc.load_gather(tdk, [rows, jnp.full((16,), bcol, jnp.int32)])
            mvk[pl.ds(j * 16, 16)] = jnp.maximum(g * a + b, 0.0)
        pltpu.sync_copy(mvk, acc.at[dvk], add=True)

    for src_h, dst_h, acol, gcol, bcol in ((srcp, dstp, 1, 3, 4),
                                           (srcc, dstc, 2, 5, 6)):
        base = wid * EPT

        def chunk_loop(j, carry):
            run_chunk(src_h, dst_h, acol, gcol, bcol, base + j * CH, CH,
                      sv, dv, tsv, tdv, mv)
            return carry

        lax.fori_loop(0, NFULL, chunk_loop, 0)
        run_chunk(src_h, dst_h, acol, gcol, bcol, base + NFULL * CH, TAIL,
                  svt, dvt, tsvt, tdvt, mvt)

    plsc.subcore_barrier()
    pltpu.sync_copy(acc.at[pl.ds(s * RPT1, RPT1)],
                    out.at[c, pl.ds(s * RPT1, RPT1)])


def _mlp_body(s1r, p0r, p1r, w0, b0, w1, b1, w2, b2, w3, b3, w4, b4, w5, b5,
              out, acc):
    k = pl.program_id(0)

    @pl.when(k == 0)
    def _():
        acc[...] = jnp.zeros((1, 128), jnp.float32)

    v = (s1r[pl.ds(k, 1), :] + p0r[pl.ds(k, 1), :] + p1r[pl.ds(k, 1), :])
    acc[...] += jnp.dot(v, w0[0], preferred_element_type=jnp.float32)

    @pl.when(k == GRID - 1)
    def _():
        x = jnp.maximum(acc[...] + b0[...], 0.0)
        for w, b in ((w1, b1), (w2, b2), (w3, b3), (w4, b4)):
            x = jnp.maximum(
                jnp.dot(x, w[...], preferred_element_type=jnp.float32)
                + b[...], 0.0)
        out[...] = (jnp.dot(x, w5[...], preferred_element_type=jnp.float32)
                    + b5[...])


def _mlp(s1r, p0r, p1r, w0, b0, w1, b1, w2, b2, w3, b3, w4, b4, w5, b5):
    row = pl.BlockSpec((GRID, RB), lambda k: (0, 0))
    full = lambda shape: pl.BlockSpec(shape, lambda k: (0, 0))
    return pl.pallas_call(
        _mlp_body,
        grid=(GRID,),
        in_specs=[
            row, row, row,
            pl.BlockSpec((1, RB, 128), lambda k: (k, 0, 0)),
            full((1, 128)),
            full((128, 128)), full((1, 128)),
            full((128, 128)), full((1, 128)),
            full((128, 128)), full((1, 128)),
            full((128, 128)), full((1, 128)),
            full((128, 1)), full((1, 1)),
        ],
        out_specs=pl.BlockSpec((1, 1), lambda k: (0, 0)),
        out_shape=jax.ShapeDtypeStruct((1, 1), jnp.float32),
        scratch_shapes=[pltpu.VMEM((1, 128), jnp.float32)],
    )(s1r, p0r, p1r, w0, b0, w1, b1, w2, b2, w3, b3, w4, b4, w5, b5)


def kernel(node_features, edges_power, edges_comm, gnn_Wself_0,
           gnn_Wrel_0_power, gnn_Wfilm_0_power, gnn_Wrel_0_comm,
           gnn_Wfilm_0_comm, gnn_Wself_1, gnn_Wrel_1_power,
           gnn_Wfilm_1_power, gnn_Wrel_1_comm, gnn_Wfilm_1_comm,
           mlp_W0, mlp_b0, mlp_W1, mlp_b1, mlp_W2, mlp_b2, mlp_W3, mlp_b3,
           mlp_W4, mlp_b4, mlp_W5, mlp_b5):
    srcp, dstp = edges_power[0], edges_power[1]
    srcc, dstc = edges_comm[0], edges_comm[1]

    inv = (1.0 / jnp.array([4.0, 1.0, 2.0, 1.0, 230.0],
                           dtype=jnp.float32))[:, None]
    s0, a_p, gb_p, a_c, gb_c = _prep0(
        node_features, gnn_Wself_0 * inv, gnn_Wrel_0_power * inv,
        gnn_Wfilm_0_power * inv, gnn_Wrel_0_comm * inv,
        gnn_Wfilm_0_comm * inv)

    z0 = jnp.zeros((N, HID), jnp.float32)
    parts0 = _edges0(srcp, dstp, srcc, dstc, a_p, gb_p, a_c, gb_c, z0)

    wt = jnp.concatenate(
        [gnn_Wself_1, gnn_Wrel_1_power, gnn_Wrel_1_comm,
         gnn_Wfilm_1_power, gnn_Wfilm_1_comm,
         jnp.zeros((HID, 9), jnp.float32)], axis=1)
    tbl, s1 = _prep1(s0, parts0, wt)

    z1 = jnp.zeros((NP1,), jnp.float32)
    parts1 = _edges1(srcp, dstp, srcc, dstc, tbl, z1)

    s1r = s1.reshape(GRID, RB)
    p0r = parts1[0, :N].reshape(GRID, RB)
    p1r = parts1[1, :N].reshape(GRID, RB)
    res = _mlp(s1r, p0r, p1r, mlp_W0.reshape(GRID, RB, 128),
               mlp_b0.reshape(1, 128),
               mlp_W1, mlp_b1.reshape(1, 128), mlp_W2, mlp_b2.reshape(1, 128),
               mlp_W3, mlp_b3.reshape(1, 128), mlp_W4, mlp_b4.reshape(1, 128),
               mlp_W5, mlp_b5.reshape(1, 1))
    return res.reshape(1)


# speedup vs baseline: 16.4724x; 1.6405x over previous
import functools

import jax
import jax.numpy as jnp
from jax import lax
from jax.experimental import pallas as pl
from jax.experimental.pallas import tpu as pltpu
from jax.experimental.pallas import tpu_sc as plsc

N = 50000
E = 1600000
HID = 32
HH = HID // 2
NC = 2
NS = 16
NW = NC * NS
EPT = E // NW
CH = 128
ROWS0 = 3128
ROWS0_LAST = N - 15 * ROWS0
NP1 = 51200
RPT1 = NP1 // NS
RB = 2000
GRID = N // RB

CH2 = 256
NCH2 = EPT // CH2
TAIL = EPT - NCH2 * CH2
NB = 3
NOUT = NCH2 // NB

_mesh = plsc.VectorSubcoreMesh(core_axis_name="c", subcore_axis_name="s")


def _prep0_body(nf, wself, wp0, wp1, fp0, fp1, wc0, wc1, fc0, fc1,
                s0, ap0, ap1, gbp0, gbp1, ac0, ac1, gbc0, gbc1):
    h = nf[...]
    dot = lambda w: jnp.dot(h, w[...], preferred_element_type=jnp.float32)
    s0[...] = dot(wself)
    ap0[...] = dot(wp0)
    ap1[...] = dot(wp1)
    gbp0[...] = dot(fp0)
    gbp1[...] = dot(fp1)
    ac0[...] = dot(wc0)
    ac1[...] = dot(wc1)
    gbc0[...] = dot(fc0)
    gbc1[...] = dot(fc1)


def _prep0(nf, wself, wp0, wp1, fp0, fp1, wc0, wc1, fc0, fc1):
    full = lambda shape: pl.BlockSpec(shape, lambda k: (0, 0))
    a_spec = pl.BlockSpec((RB, HH), lambda k: (k, 0))
    gb_spec = pl.BlockSpec((RB, HID), lambda k: (k, 0))
    a_shape = jax.ShapeDtypeStruct((N, HH), jnp.float32)
    gb_shape = jax.ShapeDtypeStruct((N, HID), jnp.float32)
    return pl.pallas_call(
        _prep0_body,
        grid=(GRID,),
        in_specs=[
            pl.BlockSpec((RB, 5), lambda k: (k, 0)),
            full((5, HID)),
            full((5, HH)), full((5, HH)), full((5, HID)), full((5, HID)),
            full((5, HH)), full((5, HH)), full((5, HID)), full((5, HID)),
        ],
        out_specs=[
            pl.BlockSpec((RB, HID), lambda k: (k, 0)),
            a_spec, a_spec, gb_spec, gb_spec,
            a_spec, a_spec, gb_spec, gb_spec,
        ],
        out_shape=[
            jax.ShapeDtypeStruct((N, HID), jnp.float32),
            a_shape, a_shape, gb_shape, gb_shape,
            a_shape, a_shape, gb_shape, gb_shape,
        ],
    )(nf, wself, wp0, wp1, fp0, fp1, wc0, wc1, fc0, fc1)


@functools.partial(
    pl.kernel,
    out_type=jax.ShapeDtypeStruct((NC, 2, N, HH), jnp.float32),
    mesh=_mesh,
    scratch_types=[
        pltpu.VMEM((NB, CH2), jnp.int32),
        pltpu.VMEM((NB, CH2), jnp.int32),
        pltpu.VMEM((NB, 2, CH), jnp.int32),
        pltpu.VMEM((NB, CH2, HH), jnp.float32),
        pltpu.VMEM((NB, CH2, HID), jnp.float32),
        pltpu.VMEM((NB, CH2, HH), jnp.float32),
        pltpu.VMEM((TAIL,), jnp.int32),
        pltpu.VMEM((TAIL,), jnp.int32),
        pltpu.VMEM((TAIL, HH), jnp.float32),
        pltpu.VMEM((TAIL, HID), jnp.float32),
        pltpu.VMEM((TAIL, HH), jnp.float32),
        pltpu.VMEM_SHARED((N, HH), jnp.float32),
        pltpu.SemaphoreType.DMA,
        pltpu.SemaphoreType.DMA,
        pltpu.SemaphoreType.DMA,
        pltpu.SemaphoreType.DMA,
        pltpu.SemaphoreType.DMA,
        pltpu.SemaphoreType.DMA,
        pltpu.SemaphoreType.DMA,
        pltpu.SemaphoreType.DMA,
        pltpu.SemaphoreType.DMA,
        pltpu.SemaphoreType.DMA,
    ],
    compiler_params=pltpu.CompilerParams(use_tc_tiling_on_sc=False),
)
def _edges0(srcp, dstp, srcc, dstc, ap0, ap1, gbp0, gbp1, ac0, ac1,
            gbc0, gbc1, z0, out,
            sv3, dv3, dvS, av3, gbv3, mv3, svt, dvt, avt, gbvt, mvt, acc,
            g0s, g1s, g2s, s0s, s1s, s2s, i0s, i1s, i2s, tsem):
    gsem = (g0s, g1s, g2s)
    ssem = (s0s, s1s, s2s)
    isem = (i0s, i1s, i2s)
    c = lax.axis_index("c")
    s = lax.axis_index("s")
    wid = c * NS + s
    base = wid * EPT

    def zero_acc():
        @pl.when(s < NS - 1)
        def _():
            pltpu.sync_copy(z0.at[pl.ds(s * ROWS0, ROWS0)],
                            acc.at[pl.ds(s * ROWS0, ROWS0)])

        @pl.when(s == NS - 1)
        def _():
            pltpu.sync_copy(z0.at[pl.ds((NS - 1) * ROWS0, ROWS0_LAST)],
                            acc.at[pl.ds((NS - 1) * ROWS0, ROWS0_LAST)])

    def run_relation(src_h, dst_h, a_h, gb_h):
        def fire_gathers(b):
            for h in range(2):
                pltpu.async_copy(a_h.at[sv3.at[b, pl.ds(h * CH, CH)]],
                                 av3.at[b, pl.ds(h * CH, CH)], gsem[b])
                pltpu.async_copy(gb_h.at[dv3.at[b, pl.ds(h * CH, CH)]],
                                 gbv3.at[b, pl.ds(h * CH, CH)], gsem[b])

        def wait_gathers(b):
            for h in range(2):
                pltpu.make_async_copy(
                    a_h.at[sv3.at[b, pl.ds(h * CH, CH)]],
                    av3.at[b, pl.ds(h * CH, CH)], gsem[b]).wait()
                pltpu.make_async_copy(
                    gb_h.at[dv3.at[b, pl.ds(h * CH, CH)]],
                    gbv3.at[b, pl.ds(h * CH, CH)], gsem[b]).wait()

        def fire_scatters(b):
            for h in range(2):
                pltpu.async_copy(mv3.at[b, pl.ds(h * CH, CH)],
                                 acc.at[dvS.at[b, h]], ssem[b], add=True)

        def wait_scatters(b):
            for h in range(2):
                pltpu.make_async_copy(mv3.at[b, pl.ds(h * CH, CH)],
                                      acc.at[dvS.at[b, h]], ssem[b]).wait()

        for b in range(NB):
            off = base + b * CH2
            pltpu.sync_copy(src_h.at[pl.ds(off, CH2)], sv3.at[b])
            pltpu.sync_copy(dst_h.at[pl.ds(off, CH2)], dv3.at[b])
            fire_gathers(b)

        def outer(g, carry):
            for b in range(NB):
                off_next = base + ((g + 1) * NB + b) * CH2
                wait_gathers(b)

                @pl.when(g > 0)
                def _():
                    wait_scatters(b)

                for t in range(16):
                    dvS[b, t // 8, pl.ds((t % 8) * 16, 16)] = (
                        dv3[b, pl.ds(t * 16, 16)])

                @pl.when(g < NOUT - 1)
                def _():
                    pltpu.async_copy(src_h.at[pl.ds(off_next, CH2)],
                                     sv3.at[b], isem[b])
                    pltpu.async_copy(dst_h.at[pl.ds(off_next, CH2)],
                                     dv3.at[b], isem[b])

                @plsc.parallel_loop(0, CH2, 1, unroll=8)
                def _(i):
                    a0 = av3[b, i, pl.ds(0, 16)]
                    gg = gbv3[b, i, pl.ds(0, 16)]
                    bb = gbv3[b, i, pl.ds(16, 16)]
                    mv3[b, i, pl.ds(0, 16)] = jnp.maximum(gg * a0 + bb, 0.0)

                fire_scatters(b)

                @pl.when(g < NOUT - 1)
                def _():
                    pltpu.make_async_copy(src_h.at[pl.ds(off_next, CH2)],
                                          sv3.at[b], isem[b]).wait()
                    pltpu.make_async_copy(dst_h.at[pl.ds(off_next, CH2)],
                                          dv3.at[b], isem[b]).wait()
                    fire_gathers(b)

            return carry

        lax.fori_loop(0, NOUT, outer, 0)
        for b in range(NB):
            wait_scatters(b)

        off = base + NCH2 * CH2
        pltpu.sync_copy(src_h.at[pl.ds(off, TAIL)], svt)
        pltpu.sync_copy(dst_h.at[pl.ds(off, TAIL)], dvt)
        cp1 = pltpu.async_copy(a_h.at[svt], avt, tsem)
        cp2 = pltpu.async_copy(gb_h.at[dvt], gbvt, tsem)
        cp1.wait()
        cp2.wait()

        def tail_body(i, carry):
            a0 = avt[i, pl.ds(0, 16)]
            gg = gbvt[i, pl.ds(0, 16)]
            bb = gbvt[i, pl.ds(16, 16)]
            mvt[i, pl.ds(0, 16)] = jnp.maximum(gg * a0 + bb, 0.0)
            return carry

        lax.fori_loop(0, TAIL, tail_body, 0)
        pltpu.sync_copy(mvt, acc.at[dvt], add=True)

    for half, tabs in enumerate((((srcp, dstp, ap0, gbp0),
                                  (srcc, dstc, ac0, gbc0)),
                                 ((srcp, dstp, ap1, gbp1),
                                  (srcc, dstc, ac1, gbc1)))):
        zero_acc()
        plsc.subcore_barrier()
        for rel in tabs:
            run_relation(*rel)
        plsc.subcore_barrier()

        @pl.when(s < NS - 1)
        def _():
            pltpu.sync_copy(acc.at[pl.ds(s * ROWS0, ROWS0)],
                            out.at[c, half, pl.ds(s * ROWS0, ROWS0)])

        @pl.when(s == NS - 1)
        def _():
            pltpu.sync_copy(acc.at[pl.ds((NS - 1) * ROWS0, ROWS0_LAST)],
                            out.at[c, half,
                                   pl.ds((NS - 1) * ROWS0, ROWS0_LAST)])

        plsc.subcore_barrier()


def _prep1_body(s0, parts, wt, t_out, s1_out):
    p = parts[...]
    lo = p[0, 0] + p[1, 0]
    hi = p[0, 1] + p[1, 1]
    h1 = jnp.maximum(s0[...] + jnp.concatenate([lo, hi], axis=-1), 0.0)
    t = jnp.dot(h1, wt[...], preferred_element_type=jnp.float32)
    t_out[...] = t
    s1_out[...] = t[:, 0:1]


def _prep1(s0, parts, wt):
    return pl.pallas_call(
        _prep1_body,
        grid=(GRID,),
        in_specs=[
            pl.BlockSpec((RB, HID), lambda k: (k, 0)),
            pl.BlockSpec((NC, 2, RB, HH), lambda k: (0, 0, k, 0)),
            pl.BlockSpec((HID, 16), lambda k: (0, 0)),
        ],
        out_specs=[
            pl.BlockSpec((RB, 16), lambda k: (k, 0)),
            pl.BlockSpec((RB, 1), lambda k: (k, 0)),
        ],
        out_shape=[
            jax.ShapeDtypeStruct((N, 16), jnp.float32),
            jax.ShapeDtypeStruct((N, 1), jnp.float32),
        ],
    )(s0, parts, wt)


@functools.partial(
    pl.kernel,
    out_type=jax.ShapeDtypeStruct((NC, NP1), jnp.float32),
    mesh=_mesh,
    scratch_types=[
        pltpu.VMEM((CH,), jnp.int32),
        pltpu.VMEM((CH,), jnp.int32),
        pltpu.VMEM((CH, 16), jnp.float32),
        pltpu.VMEM((CH, 16), jnp.float32),
        pltpu.VMEM((CH,), jnp.float32),
        pltpu.VMEM((TAIL,), jnp.int32),
        pltpu.VMEM((TAIL,), jnp.int32),
        pltpu.VMEM((TAIL, 16), jnp.float32),
        pltpu.VMEM((TAIL, 16), jnp.float32),
        pltpu.VMEM((TAIL,), jnp.float32),
        pltpu.VMEM_SHARED((NP1,), jnp.float32),
        pltpu.SemaphoreType.DMA,
    ],
    compiler_params=pltpu.CompilerParams(use_tc_tiling_on_sc=False,
                                         needs_layout_passes=False),
)
def _edges1(srcp, dstp, srcc, dstc, tbl, z1, out,
            sv, dv, tsv, tdv, mv, svt, dvt, tsvt, tdvt, mvt, acc, sem):
    c = lax.axis_index("c")
    s = lax.axis_index("s")
    wid = c * NS + s
    pltpu.sync_copy(z1.at[pl.ds(s * RPT1, RPT1)],
                    acc.at[pl.ds(s * RPT1, RPT1)])
    plsc.subcore_barrier()

    def run_chunk(src_h, dst_h, acol, gcol, bcol, off, k,
                  svk, dvk, tsk, tdk, mvk):
        pltpu.sync_copy(src_h.at[pl.ds(off, k)], svk)
        pltpu.sync_copy(dst_h.at[pl.ds(off, k)], dvk)
        cp1 = pltpu.async_copy(tbl.at[svk], tsk, sem)
        cp2 = pltpu.async_copy(tbl.at[dvk], tdk, sem)
        cp1.wait()
        cp2.wait()
        iot = lax.iota(jnp.int32, 16)
        for j in range(k // 16):
            rows = j * 16 + iot
            a = plsc.load_gather(tsk, [rows, jnp.full((16,), acol, jnp.int32)])
            g = plsc.load_gather(tdk, [rows, jnp.full((16,), gcol, jnp.int32)])
            b = plsc.load_gather(tdk, [rows, jnp.full((16,), bcol, jnp.int32)])
            mvk[pl.ds(j * 16, 16)] = jnp.maximum(g * a + b, 0.0)
        pltpu.sync_copy(mvk, acc.at[dvk], add=True)

    for src_h, dst_h, acol, gcol, bcol in ((srcp, dstp, 1, 3, 4),
                                           (srcc, dstc, 2, 5, 6)):
        base = wid * EPT

        def chunk_loop(j, carry):
            run_chunk(src_h, dst_h, acol, gcol, bcol, base + j * CH, CH,
                      sv, dv, tsv, tdv, mv)
            return carry

        lax.fori_loop(0, EPT // CH, chunk_loop, 0)
        run_chunk(src_h, dst_h, acol, gcol, bcol, base + (EPT // CH) * CH,
                  TAIL, svt, dvt, tsvt, tdvt, mvt)

    plsc.subcore_barrier()
    pltpu.sync_copy(acc.at[pl.ds(s * RPT1, RPT1)],
                    out.at[c, pl.ds(s * RPT1, RPT1)])


def _mlp_body(s1r, p0r, p1r, w0, b0, w1, b1, w2, b2, w3, b3, w4, b4, w5, b5,
              out, acc):
    k = pl.program_id(0)

    @pl.when(k == 0)
    def _():
        acc[...] = jnp.zeros((1, 128), jnp.float32)

    v = (s1r[pl.ds(k, 1), :] + p0r[pl.ds(k, 1), :] + p1r[pl.ds(k, 1), :])
    acc[...] += jnp.dot(v, w0[0], preferred_element_type=jnp.float32)

    @pl.when(k == GRID - 1)
    def _():
        x = jnp.maximum(acc[...] + b0[...], 0.0)
        for w, b in ((w1, b1), (w2, b2), (w3, b3), (w4, b4)):
            x = jnp.maximum(
                jnp.dot(x, w[...], preferred_element_type=jnp.float32)
                + b[...], 0.0)
        out[...] = (jnp.dot(x, w5[...], preferred_element_type=jnp.float32)
                    + b5[...])


def _mlp(s1r, p0r, p1r, w0, b0, w1, b1, w2, b2, w3, b3, w4, b4, w5, b5):
    row = pl.BlockSpec((GRID, RB), lambda k: (0, 0))
    full = lambda shape: pl.BlockSpec(shape, lambda k: (0, 0))
    return pl.pallas_call(
        _mlp_body,
        grid=(GRID,),
        in_specs=[
            row, row, row,
            pl.BlockSpec((1, RB, 128), lambda k: (k, 0, 0)),
            full((1, 128)),
            full((128, 128)), full((1, 128)),
            full((128, 128)), full((1, 128)),
            full((128, 128)), full((1, 128)),
            full((128, 128)), full((1, 128)),
            full((128, 1)), full((1, 1)),
        ],
        out_specs=pl.BlockSpec((1, 1), lambda k: (0, 0)),
        out_shape=jax.ShapeDtypeStruct((1, 1), jnp.float32),
        scratch_shapes=[pltpu.VMEM((1, 128), jnp.float32)],
    )(s1r, p0r, p1r, w0, b0, w1, b1, w2, b2, w3, b3, w4, b4, w5, b5)


def kernel(node_features, edges_power, edges_comm, gnn_Wself_0,
           gnn_Wrel_0_power, gnn_Wfilm_0_power, gnn_Wrel_0_comm,
           gnn_Wfilm_0_comm, gnn_Wself_1, gnn_Wrel_1_power,
           gnn_Wfilm_1_power, gnn_Wrel_1_comm, gnn_Wfilm_1_comm,
           mlp_W0, mlp_b0, mlp_W1, mlp_b1, mlp_W2, mlp_b2, mlp_W3, mlp_b3,
           mlp_W4, mlp_b4, mlp_W5, mlp_b5):
    srcp, dstp = edges_power[0], edges_power[1]
    srcc, dstc = edges_comm[0], edges_comm[1]

    inv = (1.0 / jnp.array([4.0, 1.0, 2.0, 1.0, 230.0],
                           dtype=jnp.float32))[:, None]
    wrp = gnn_Wrel_0_power * inv
    wrc = gnn_Wrel_0_comm * inv
    wfp = gnn_Wfilm_0_power * inv
    wfc = gnn_Wfilm_0_comm * inv
    fp0 = jnp.concatenate([wfp[:, 0:HH], wfp[:, HID:HID + HH]], axis=1)
    fp1 = jnp.concatenate([wfp[:, HH:HID], wfp[:, HID + HH:]], axis=1)
    fc0 = jnp.concatenate([wfc[:, 0:HH], wfc[:, HID:HID + HH]], axis=1)
    fc1 = jnp.concatenate([wfc[:, HH:HID], wfc[:, HID + HH:]], axis=1)

    s0, ap0, ap1, gbp0, gbp1, ac0, ac1, gbc0, gbc1 = _prep0(
        node_features, gnn_Wself_0 * inv, wrp[:, :HH], wrp[:, HH:],
        fp0, fp1, wrc[:, :HH], wrc[:, HH:], fc0, fc1)

    z0 = jnp.zeros((N, HH), jnp.float32)
    parts0 = _edges0(srcp, dstp, srcc, dstc, ap0, ap1, gbp0, gbp1,
                     ac0, ac1, gbc0, gbc1, z0)

    wt = jnp.concatenate(
        [gnn_Wself_1, gnn_Wrel_1_power, gnn_Wrel_1_comm,
         gnn_Wfilm_1_power, gnn_Wfilm_1_comm,
         jnp.zeros((HID, 9), jnp.float32)], axis=1)
    tbl, s1 = _prep1(s0, parts0, wt)

    z1 = jnp.zeros((NP1,), jnp.float32)
    parts1 = _edges1(srcp, dstp, srcc, dstc, tbl, z1)

    s1r = s1.reshape(GRID, RB)
    p0r = parts1[0, :N].reshape(GRID, RB)
    p1r = parts1[1, :N].reshape(GRID, RB)
    res = _mlp(s1r, p0r, p1r, mlp_W0.reshape(GRID, RB, 128),
               mlp_b0.reshape(1, 128),
               mlp_W1, mlp_b1.reshape(1, 128), mlp_W2, mlp_b2.reshape(1, 128),
               mlp_W3, mlp_b3.reshape(1, 128), mlp_W4, mlp_b4.reshape(1, 128),
               mlp_W5, mlp_b5.reshape(1, 1))
    return res.reshape(1)


# speedup vs baseline: 18.0211x; 1.0940x over previous
import functools

import jax
import jax.numpy as jnp
from jax import lax
from jax.experimental import pallas as pl
from jax.experimental.pallas import tpu as pltpu
from jax.experimental.pallas import tpu_sc as plsc

N = 50000
E = 1600000
HID = 32
HH = HID // 2
NC = 2
NS = 16
NW = NC * NS
EPT = E // NW
CH = 128
ROWS0 = 3128
ROWS0_LAST = N - 15 * ROWS0
NP1 = 51200
RPT1 = NP1 // NS
RB = 2000
GRID = N // RB

CH2 = 256
NCH2 = EPT // CH2
TAIL = EPT - NCH2 * CH2
NB = 3
NOUT = NCH2 // NB

_mesh = plsc.VectorSubcoreMesh(core_axis_name="c", subcore_axis_name="s")


def _prep0_body(nf, wself, wp0, wp1, fp0, fp1, wc0, wc1, fc0, fc1,
                s0, ap0, ap1, gbp0, gbp1, ac0, ac1, gbc0, gbc1):
    h = nf[...]
    dot = lambda w: jnp.dot(h, w[...], preferred_element_type=jnp.float32)
    s0[...] = dot(wself)
    ap0[...] = dot(wp0)
    ap1[...] = dot(wp1)
    gbp0[...] = dot(fp0)
    gbp1[...] = dot(fp1)
    ac0[...] = dot(wc0)
    ac1[...] = dot(wc1)
    gbc0[...] = dot(fc0)
    gbc1[...] = dot(fc1)


def _prep0(nf, wself, wp0, wp1, fp0, fp1, wc0, wc1, fc0, fc1):
    full = lambda shape: pl.BlockSpec(shape, lambda k: (0, 0))
    a_spec = pl.BlockSpec((RB, HH), lambda k: (k, 0))
    gb_spec = pl.BlockSpec((RB, HID), lambda k: (k, 0))
    a_shape = jax.ShapeDtypeStruct((N, HH), jnp.float32)
    gb_shape = jax.ShapeDtypeStruct((N, HID), jnp.float32)
    return pl.pallas_call(
        _prep0_body,
        grid=(GRID,),
        in_specs=[
            pl.BlockSpec((RB, 5), lambda k: (k, 0)),
            full((5, HID)),
            full((5, HH)), full((5, HH)), full((5, HID)), full((5, HID)),
            full((5, HH)), full((5, HH)), full((5, HID)), full((5, HID)),
        ],
        out_specs=[
            pl.BlockSpec((RB, HID), lambda k: (k, 0)),
            a_spec, a_spec, gb_spec, gb_spec,
            a_spec, a_spec, gb_spec, gb_spec,
        ],
        out_shape=[
            jax.ShapeDtypeStruct((N, HID), jnp.float32),
            a_shape, a_shape, gb_shape, gb_shape,
            a_shape, a_shape, gb_shape, gb_shape,
        ],
    )(nf, wself, wp0, wp1, fp0, fp1, wc0, wc1, fc0, fc1)


@functools.partial(
    pl.kernel,
    out_type=jax.ShapeDtypeStruct((NC, 2, N, HH), jnp.float32),
    mesh=_mesh,
    scratch_types=[
        pltpu.VMEM((NB, 2, CH), jnp.int32),
        pltpu.VMEM((NB, 2, CH), jnp.int32),
        pltpu.VMEM((NB, 2, CH), jnp.int32),
        pltpu.VMEM((NB, CH2, HH), jnp.float32),
        pltpu.VMEM((NB, CH2, HID), jnp.float32),
        pltpu.VMEM((NB, CH2, HH), jnp.float32),
        pltpu.VMEM((TAIL,), jnp.int32),
        pltpu.VMEM((TAIL,), jnp.int32),
        pltpu.VMEM((TAIL, HH), jnp.float32),
        pltpu.VMEM((TAIL, HID), jnp.float32),
        pltpu.VMEM((TAIL, HH), jnp.float32),
        pltpu.VMEM_SHARED((N, HH), jnp.float32),
        pltpu.SemaphoreType.DMA,
        pltpu.SemaphoreType.DMA,
        pltpu.SemaphoreType.DMA,
        pltpu.SemaphoreType.DMA,
        pltpu.SemaphoreType.DMA,
        pltpu.SemaphoreType.DMA,
        pltpu.SemaphoreType.DMA,
        pltpu.SemaphoreType.DMA,
        pltpu.SemaphoreType.DMA,
        pltpu.SemaphoreType.DMA,
    ],
    compiler_params=pltpu.CompilerParams(use_tc_tiling_on_sc=False),
)
def _edges0(srcp, dstp, srcc, dstc, ap0, ap1, gbp0, gbp1, ac0, ac1,
            gbc0, gbc1, z0, out,
            sv3, dv3, dvS, av3, gbv3, mv3, svt, dvt, avt, gbvt, mvt, acc,
            g0s, g1s, g2s, s0s, s1s, s2s, i0s, i1s, i2s, tsem):
    gsem = (g0s, g1s, g2s)
    ssem = (s0s, s1s, s2s)
    isem = (i0s, i1s, i2s)
    c = lax.axis_index("c")
    s = lax.axis_index("s")
    wid = c * NS + s
    base = wid * EPT

    def zero_acc():
        @pl.when(s < NS - 1)
        def _():
            pltpu.sync_copy(z0.at[pl.ds(s * ROWS0, ROWS0)],
                            acc.at[pl.ds(s * ROWS0, ROWS0)])

        @pl.when(s == NS - 1)
        def _():
            pltpu.sync_copy(z0.at[pl.ds((NS - 1) * ROWS0, ROWS0_LAST)],
                            acc.at[pl.ds((NS - 1) * ROWS0, ROWS0_LAST)])

    def run_relation(src_h, dst_h, a_h, gb_h):
        def fire_gathers(b):
            for h in range(2):
                pltpu.async_copy(a_h.at[sv3.at[b, h]],
                                 av3.at[b, pl.ds(h * CH, CH)], gsem[b])
                pltpu.async_copy(gb_h.at[dv3.at[b, h]],
                                 gbv3.at[b, pl.ds(h * CH, CH)], gsem[b])

        def wait_gathers(b):
            for h in range(2):
                pltpu.make_async_copy(
                    a_h.at[sv3.at[b, h]],
                    av3.at[b, pl.ds(h * CH, CH)], gsem[b]).wait()
                pltpu.make_async_copy(
                    gb_h.at[dv3.at[b, h]],
                    gbv3.at[b, pl.ds(h * CH, CH)], gsem[b]).wait()

        def fire_scatters(b):
            for h in range(2):
                pltpu.async_copy(mv3.at[b, pl.ds(h * CH, CH)],
                                 acc.at[dvS.at[b, h]], ssem[b], add=True)

        def wait_scatters(b):
            for h in range(2):
                pltpu.make_async_copy(mv3.at[b, pl.ds(h * CH, CH)],
                                      acc.at[dvS.at[b, h]], ssem[b]).wait()

        for b in range(NB):
            off = base + b * CH2
            for h in range(2):
                pltpu.sync_copy(src_h.at[pl.ds(off + h * CH, CH)],
                                sv3.at[b, h])
                pltpu.sync_copy(dst_h.at[pl.ds(off + h * CH, CH)],
                                dv3.at[b, h])
            fire_gathers(b)

        def outer(g, carry):
            for b in range(NB):
                off_next = base + ((g + 1) * NB + b) * CH2
                wait_gathers(b)

                @pl.when(g > 0)
                def _():
                    wait_scatters(b)

                for t in range(16):
                    dvS[b, t // 8, pl.ds((t % 8) * 16, 16)] = (
                        dv3[b, t // 8, pl.ds((t % 8) * 16, 16)])

                @pl.when(g < NOUT - 1)
                def _():
                    for h in range(2):
                        pltpu.async_copy(
                            src_h.at[pl.ds(off_next + h * CH, CH)],
                            sv3.at[b, h], isem[b])
                        pltpu.async_copy(
                            dst_h.at[pl.ds(off_next + h * CH, CH)],
                            dv3.at[b, h], isem[b])

                def cbody(i, carry):
                    a0 = av3[b, i, pl.ds(0, 16)]
                    gg = gbv3[b, i, pl.ds(0, 16)]
                    bb = gbv3[b, i, pl.ds(16, 16)]
                    mv3[b, i, pl.ds(0, 16)] = jnp.maximum(gg * a0 + bb, 0.0)
                    return carry

                lax.fori_loop(0, CH2, cbody, 0)

                fire_scatters(b)

                @pl.when(g < NOUT - 1)
                def _():
                    for h in range(2):
                        pltpu.make_async_copy(
                            src_h.at[pl.ds(off_next + h * CH, CH)],
                            sv3.at[b, h], isem[b]).wait()
                        pltpu.make_async_copy(
                            dst_h.at[pl.ds(off_next + h * CH, CH)],
                            dv3.at[b, h], isem[b]).wait()
                    fire_gathers(b)

            return carry

        lax.fori_loop(0, NOUT, outer, 0)
        for b in range(NB):
            wait_scatters(b)

        off = base + NCH2 * CH2
        pltpu.sync_copy(src_h.at[pl.ds(off, TAIL)], svt)
        pltpu.sync_copy(dst_h.at[pl.ds(off, TAIL)], dvt)
        cp1 = pltpu.async_copy(a_h.at[svt], avt, tsem)
        cp2 = pltpu.async_copy(gb_h.at[dvt], gbvt, tsem)
        cp1.wait()
        cp2.wait()

        def tail_body(i, carry):
            a0 = avt[i, pl.ds(0, 16)]
            gg = gbvt[i, pl.ds(0, 16)]
            bb = gbvt[i, pl.ds(16, 16)]
            mvt[i, pl.ds(0, 16)] = jnp.maximum(gg * a0 + bb, 0.0)
            return carry

        lax.fori_loop(0, TAIL, tail_body, 0)
        pltpu.sync_copy(mvt, acc.at[dvt], add=True)

    for half, tabs in enumerate((((srcp, dstp, ap0, gbp0),
                                  (srcc, dstc, ac0, gbc0)),
                                 ((srcp, dstp, ap1, gbp1),
                                  (srcc, dstc, ac1, gbc1)))):
        zero_acc()
        plsc.subcore_barrier()
        for rel in tabs:
            run_relation(*rel)
        plsc.subcore_barrier()

        @pl.when(s < NS - 1)
        def _():
            pltpu.sync_copy(acc.at[pl.ds(s * ROWS0, ROWS0)],
                            out.at[c, half, pl.ds(s * ROWS0, ROWS0)])

        @pl.when(s == NS - 1)
        def _():
            pltpu.sync_copy(acc.at[pl.ds((NS - 1) * ROWS0, ROWS0_LAST)],
                            out.at[c, half,
                                   pl.ds((NS - 1) * ROWS0, ROWS0_LAST)])

        plsc.subcore_barrier()


def _prep1_body(s0, parts, wt, t_out, s1_out):
    p = parts[...]
    lo = p[0, 0] + p[1, 0]
    hi = p[0, 1] + p[1, 1]
    h1 = jnp.maximum(s0[...] + jnp.concatenate([lo, hi], axis=-1), 0.0)
    t = jnp.dot(h1, wt[...], preferred_element_type=jnp.float32)
    t_out[...] = t
    s1_out[...] = t[:, 0:1]


def _prep1(s0, parts, wt):
    return pl.pallas_call(
        _prep1_body,
        grid=(GRID,),
        in_specs=[
            pl.BlockSpec((RB, HID), lambda k: (k, 0)),
            pl.BlockSpec((NC, 2, RB, HH), lambda k: (0, 0, k, 0)),
            pl.BlockSpec((HID, 16), lambda k: (0, 0)),
        ],
        out_specs=[
            pl.BlockSpec((RB, 16), lambda k: (k, 0)),
            pl.BlockSpec((RB, 1), lambda k: (k, 0)),
        ],
        out_shape=[
            jax.ShapeDtypeStruct((N, 16), jnp.float32),
            jax.ShapeDtypeStruct((N, 1), jnp.float32),
        ],
    )(s0, parts, wt)


@functools.partial(
    pl.kernel,
    out_type=jax.ShapeDtypeStruct((NC, NP1), jnp.float32),
    mesh=_mesh,
    scratch_types=[
        pltpu.VMEM((NB, 2, CH), jnp.int32),
        pltpu.VMEM((NB, 2, CH), jnp.int32),
        pltpu.VMEM((NB, 2, CH), jnp.int32),
        pltpu.VMEM((NB * CH2,), jnp.float32),
        pltpu.VMEM((NB * CH2,), jnp.float32),
        pltpu.VMEM((NB * CH2,), jnp.float32),
        pltpu.VMEM((NB * CH2,), jnp.float32),
        pltpu.VMEM((TAIL,), jnp.int32),
        pltpu.VMEM((TAIL,), jnp.int32),
        pltpu.VMEM((TAIL,), jnp.float32),
        pltpu.VMEM((TAIL,), jnp.float32),
        pltpu.VMEM((TAIL,), jnp.float32),
        pltpu.VMEM((TAIL,), jnp.float32),
        pltpu.VMEM_SHARED((NP1,), jnp.float32),
        pltpu.SemaphoreType.DMA,
        pltpu.SemaphoreType.DMA,
        pltpu.SemaphoreType.DMA,
        pltpu.SemaphoreType.DMA,
        pltpu.SemaphoreType.DMA,
        pltpu.SemaphoreType.DMA,
        pltpu.SemaphoreType.DMA,
        pltpu.SemaphoreType.DMA,
        pltpu.SemaphoreType.DMA,
        pltpu.SemaphoreType.DMA,
    ],
    compiler_params=pltpu.CompilerParams(use_tc_tiling_on_sc=False),
)
def _edges1(srcp, dstp, srcc, dstc, a_p, g_p, b_p, a_c, g_c, b_c, z1, out,
            sv3, dv3, dvS, av3, gv3, bv3, mv3,
            svt, dvt, avt, gvt, bvt, mvt, acc,
            g0s, g1s, g2s, s0s, s1s, s2s, i0s, i1s, i2s, tsem):
    gsem = (g0s, g1s, g2s)
    ssem = (s0s, s1s, s2s)
    isem = (i0s, i1s, i2s)
    c = lax.axis_index("c")
    s = lax.axis_index("s")
    wid = c * NS + s
    base = wid * EPT
    pltpu.sync_copy(z1.at[pl.ds(s * RPT1, RPT1)],
                    acc.at[pl.ds(s * RPT1, RPT1)])
    plsc.subcore_barrier()

    def run_relation(src_h, dst_h, a_h, g_h, b_h):
        def fire_gathers(b):
            for h in range(2):
                pltpu.async_copy(a_h.at[sv3.at[b, h]],
                                 av3.at[pl.ds(b * CH2 + h * CH, CH)], gsem[b])
                pltpu.async_copy(g_h.at[dv3.at[b, h]],
                                 gv3.at[pl.ds(b * CH2 + h * CH, CH)], gsem[b])
                pltpu.async_copy(b_h.at[dv3.at[b, h]],
                                 bv3.at[pl.ds(b * CH2 + h * CH, CH)], gsem[b])

        def wait_gathers(b):
            for h in range(2):
                pltpu.make_async_copy(
                    a_h.at[sv3.at[b, h]],
                    av3.at[pl.ds(b * CH2 + h * CH, CH)], gsem[b]).wait()
                pltpu.make_async_copy(
                    g_h.at[dv3.at[b, h]],
                    gv3.at[pl.ds(b * CH2 + h * CH, CH)], gsem[b]).wait()
                pltpu.make_async_copy(
                    b_h.at[dv3.at[b, h]],
                    bv3.at[pl.ds(b * CH2 + h * CH, CH)], gsem[b]).wait()

        def fire_scatters(b):
            for h in range(2):
                pltpu.async_copy(mv3.at[pl.ds(b * CH2 + h * CH, CH)],
                                 acc.at[dvS.at[b, h]], ssem[b], add=True)

        def wait_scatters(b):
            for h in range(2):
                pltpu.make_async_copy(mv3.at[pl.ds(b * CH2 + h * CH, CH)],
                                      acc.at[dvS.at[b, h]], ssem[b]).wait()

        for b in range(NB):
            off = base + b * CH2
            for h in range(2):
                pltpu.sync_copy(src_h.at[pl.ds(off + h * CH, CH)],
                                sv3.at[b, h])
                pltpu.sync_copy(dst_h.at[pl.ds(off + h * CH, CH)],
                                dv3.at[b, h])
            fire_gathers(b)

        def outer(g, carry):
            for b in range(NB):
                off_next = base + ((g + 1) * NB + b) * CH2
                wait_gathers(b)

                @pl.when(g > 0)
                def _():
                    wait_scatters(b)

                for t in range(16):
                    dvS[b, t // 8, pl.ds((t % 8) * 16, 16)] = (
                        dv3[b, t // 8, pl.ds((t % 8) * 16, 16)])

                @pl.when(g < NOUT - 1)
                def _():
                    for h in range(2):
                        pltpu.async_copy(
                            src_h.at[pl.ds(off_next + h * CH, CH)],
                            sv3.at[b, h], isem[b])
                        pltpu.async_copy(
                            dst_h.at[pl.ds(off_next + h * CH, CH)],
                            dv3.at[b, h], isem[b])

                def cbody(j, carry):
                    i0 = b * CH2 + j * 16
                    mv3[pl.ds(i0, 16)] = jnp.maximum(
                        gv3[pl.ds(i0, 16)] * av3[pl.ds(i0, 16)]
                        + bv3[pl.ds(i0, 16)], 0.0)
                    return carry

                lax.fori_loop(0, CH2 // 16, cbody, 0)

                fire_scatters(b)

                @pl.when(g < NOUT - 1)
                def _():
                    for h in range(2):
                        pltpu.make_async_copy(
                            src_h.at[pl.ds(off_next + h * CH, CH)],
                            sv3.at[b, h], isem[b]).wait()
                        pltpu.make_async_copy(
                            dst_h.at[pl.ds(off_next + h * CH, CH)],
                            dv3.at[b, h], isem[b]).wait()
                    fire_gathers(b)

            return carry

        lax.fori_loop(0, NOUT, outer, 0)
        for b in range(NB):
            wait_scatters(b)

        off = base + NCH2 * CH2
        pltpu.sync_copy(src_h.at[pl.ds(off, TAIL)], svt)
        pltpu.sync_copy(dst_h.at[pl.ds(off, TAIL)], dvt)
        cps = [pltpu.async_copy(a_h.at[svt], avt, tsem),
               pltpu.async_copy(g_h.at[dvt], gvt, tsem),
               pltpu.async_copy(b_h.at[dvt], bvt, tsem)]
        for cp in cps:
            cp.wait()
        for j in range(TAIL // 16):
            i0 = j * 16
            mvt[pl.ds(i0, 16)] = jnp.maximum(
                gvt[pl.ds(i0, 16)] * avt[pl.ds(i0, 16)]
                + bvt[pl.ds(i0, 16)], 0.0)
        pltpu.sync_copy(mvt, acc.at[dvt], add=True)

    run_relation(srcp, dstp, a_p, g_p, b_p)
    run_relation(srcc, dstc, a_c, g_c, b_c)

    plsc.subcore_barrier()
    pltpu.sync_copy(acc.at[pl.ds(s * RPT1, RPT1)],
                    out.at[c, pl.ds(s * RPT1, RPT1)])


def _mlp_body(s1r, p0r, p1r, w0, b0, w1, b1, w2, b2, w3, b3, w4, b4, w5, b5,
              out, acc):
    k = pl.program_id(0)

    @pl.when(k == 0)
    def _():
        acc[...] = jnp.zeros((1, 128), jnp.float32)

    v = (s1r[pl.ds(k, 1), :] + p0r[pl.ds(k, 1), :] + p1r[pl.ds(k, 1), :])
    acc[...] += jnp.dot(v, w0[0], preferred_element_type=jnp.float32)

    @pl.when(k == GRID - 1)
    def _():
        x = jnp.maximum(acc[...] + b0[...], 0.0)
        for w, b in ((w1, b1), (w2, b2), (w3, b3), (w4, b4)):
            x = jnp.maximum(
                jnp.dot(x, w[...], preferred_element_type=jnp.float32)
                + b[...], 0.0)
        out[...] = (jnp.dot(x, w5[...], preferred_element_type=jnp.float32)
                    + b5[...])


def _mlp(s1r, p0r, p1r, w0, b0, w1, b1, w2, b2, w3, b3, w4, b4, w5, b5):
    row = pl.BlockSpec((GRID, RB), lambda k: (0, 0))
    full = lambda shape: pl.BlockSpec(shape, lambda k: (0, 0))
    return pl.pallas_call(
        _mlp_body,
        grid=(GRID,),
        in_specs=[
            row, row, row,
            pl.BlockSpec((1, RB, 128), lambda k: (k, 0, 0)),
            full((1, 128)),
            full((128, 128)), full((1, 128)),
            full((128, 128)), full((1, 128)),
            full((128, 128)), full((1, 128)),
            full((128, 128)), full((1, 128)),
            full((128, 1)), full((1, 1)),
        ],
        out_specs=pl.BlockSpec((1, 1), lambda k: (0, 0)),
        out_shape=jax.ShapeDtypeStruct((1, 1), jnp.float32),
        scratch_shapes=[pltpu.VMEM((1, 128), jnp.float32)],
    )(s1r, p0r, p1r, w0, b0, w1, b1, w2, b2, w3, b3, w4, b4, w5, b5)


def kernel(node_features, edges_power, edges_comm, gnn_Wself_0,
           gnn_Wrel_0_power, gnn_Wfilm_0_power, gnn_Wrel_0_comm,
           gnn_Wfilm_0_comm, gnn_Wself_1, gnn_Wrel_1_power,
           gnn_Wfilm_1_power, gnn_Wrel_1_comm, gnn_Wfilm_1_comm,
           mlp_W0, mlp_b0, mlp_W1, mlp_b1, mlp_W2, mlp_b2, mlp_W3, mlp_b3,
           mlp_W4, mlp_b4, mlp_W5, mlp_b5):
    srcp, dstp = edges_power[0], edges_power[1]
    srcc, dstc = edges_comm[0], edges_comm[1]

    inv = (1.0 / jnp.array([4.0, 1.0, 2.0, 1.0, 230.0],
                           dtype=jnp.float32))[:, None]
    wrp = gnn_Wrel_0_power * inv
    wrc = gnn_Wrel_0_comm * inv
    wfp = gnn_Wfilm_0_power * inv
    wfc = gnn_Wfilm_0_comm * inv
    fp0 = jnp.concatenate([wfp[:, 0:HH], wfp[:, HID:HID + HH]], axis=1)
    fp1 = jnp.concatenate([wfp[:, HH:HID], wfp[:, HID + HH:]], axis=1)
    fc0 = jnp.concatenate([wfc[:, 0:HH], wfc[:, HID:HID + HH]], axis=1)
    fc1 = jnp.concatenate([wfc[:, HH:HID], wfc[:, HID + HH:]], axis=1)

    s0, ap0, ap1, gbp0, gbp1, ac0, ac1, gbc0, gbc1 = _prep0(
        node_features, gnn_Wself_0 * inv, wrp[:, :HH], wrp[:, HH:],
        fp0, fp1, wrc[:, :HH], wrc[:, HH:], fc0, fc1)

    z0 = jnp.zeros((N, HH), jnp.float32)
    parts0 = _edges0(srcp, dstp, srcc, dstc, ap0, ap1, gbp0, gbp1,
                     ac0, ac1, gbc0, gbc1, z0)

    wt = jnp.concatenate(
        [gnn_Wself_1, gnn_Wrel_1_power, gnn_Wrel_1_comm,
         gnn_Wfilm_1_power, gnn_Wfilm_1_comm,
         jnp.zeros((HID, 9), jnp.float32)], axis=1)
    tbl, s1 = _prep1(s0, parts0, wt)

    z1 = jnp.zeros((NP1,), jnp.float32)
    parts1 = _edges1(srcp, dstp, srcc, dstc,
                     tbl[:, 1], tbl[:, 3], tbl[:, 4],
                     tbl[:, 2], tbl[:, 5], tbl[:, 6], z1)

    s1r = s1.reshape(GRID, RB)
    p0r = parts1[0, :N].reshape(GRID, RB)
    p1r = parts1[1, :N].reshape(GRID, RB)
    res = _mlp(s1r, p0r, p1r, mlp_W0.reshape(GRID, RB, 128),
               mlp_b0.reshape(1, 128),
               mlp_W1, mlp_b1.reshape(1, 128), mlp_W2, mlp_b2.reshape(1, 128),
               mlp_W3, mlp_b3.reshape(1, 128), mlp_W4, mlp_b4.reshape(1, 128),
               mlp_W5, mlp_b5.reshape(1, 1))
    return res.reshape(1)


# speedup vs baseline: 18.4912x; 1.0261x over previous
import functools

import jax
import jax.numpy as jnp
from jax import lax
from jax.experimental import pallas as pl
from jax.experimental.pallas import tpu as pltpu
from jax.experimental.pallas import tpu_sc as plsc

N = 50000
E = 1600000
HID = 32
HH = HID // 2
NC = 2
NS = 16
NW = NC * NS
EPT = E // NW
CH = 128
ROWS0 = 3128
ROWS0_LAST = N - 15 * ROWS0
NP1 = 51200
RPT1 = NP1 // NS
RB = 2000
GRID = N // RB

CH2 = 256
NCH2 = EPT // CH2
TAIL = EPT - NCH2 * CH2
NB = 3
NOUT = NCH2 // NB

_mesh = plsc.VectorSubcoreMesh(core_axis_name="c", subcore_axis_name="s")


def _prep0_body(nf, wself, wp0, wp1, fp0, fp1, wc0, wc1, fc0, fc1,
                s0, ap0, ap1, gbp0, gbp1, ac0, ac1, gbc0, gbc1):
    h = nf[...]
    dot = lambda w: jnp.dot(h, w[...], preferred_element_type=jnp.float32)
    s0[...] = dot(wself)
    ap0[...] = dot(wp0)
    ap1[...] = dot(wp1)
    gbp0[...] = dot(fp0)
    gbp1[...] = dot(fp1)
    ac0[...] = dot(wc0)
    ac1[...] = dot(wc1)
    gbc0[...] = dot(fc0)
    gbc1[...] = dot(fc1)


def _prep0(nf, wself, wp0, wp1, fp0, fp1, wc0, wc1, fc0, fc1):
    full = lambda shape: pl.BlockSpec(shape, lambda k: (0, 0))
    a_spec = pl.BlockSpec((RB, HH), lambda k: (k, 0))
    gb_spec = pl.BlockSpec((RB, HID), lambda k: (k, 0))
    a_shape = jax.ShapeDtypeStruct((N, HH), jnp.float32)
    gb_shape = jax.ShapeDtypeStruct((N, HID), jnp.float32)
    return pl.pallas_call(
        _prep0_body,
        grid=(GRID,),
        in_specs=[
            pl.BlockSpec((RB, 5), lambda k: (k, 0)),
            full((5, HID)),
            full((5, HH)), full((5, HH)), full((5, HID)), full((5, HID)),
            full((5, HH)), full((5, HH)), full((5, HID)), full((5, HID)),
        ],
        out_specs=[
            pl.BlockSpec((RB, HID), lambda k: (k, 0)),
            a_spec, a_spec, gb_spec, gb_spec,
            a_spec, a_spec, gb_spec, gb_spec,
        ],
        out_shape=[
            jax.ShapeDtypeStruct((N, HID), jnp.float32),
            a_shape, a_shape, gb_shape, gb_shape,
            a_shape, a_shape, gb_shape, gb_shape,
        ],
    )(nf, wself, wp0, wp1, fp0, fp1, wc0, wc1, fc0, fc1)


@functools.partial(
    pl.kernel,
    out_type=jax.ShapeDtypeStruct((NC, 2, N, HH), jnp.float32),
    mesh=_mesh,
    scratch_types=[
        pltpu.VMEM((NB, 2, CH), jnp.int32),
        pltpu.VMEM((NB, 2, CH), jnp.int32),
        pltpu.VMEM((NB, 2, CH), jnp.int32),
        pltpu.VMEM((NB, CH2, HH), jnp.float32),
        pltpu.VMEM((NB, CH2, HID), jnp.float32),
        pltpu.VMEM((NB, CH2, HH), jnp.float32),
        pltpu.VMEM((TAIL,), jnp.int32),
        pltpu.VMEM((TAIL,), jnp.int32),
        pltpu.VMEM((TAIL, HH), jnp.float32),
        pltpu.VMEM((TAIL, HID), jnp.float32),
        pltpu.VMEM((TAIL, HH), jnp.float32),
        pltpu.VMEM_SHARED((N, HH), jnp.float32),
        pltpu.SemaphoreType.DMA,
        pltpu.SemaphoreType.DMA,
        pltpu.SemaphoreType.DMA,
        pltpu.SemaphoreType.DMA,
        pltpu.SemaphoreType.DMA,
        pltpu.SemaphoreType.DMA,
        pltpu.SemaphoreType.DMA,
        pltpu.SemaphoreType.DMA,
        pltpu.SemaphoreType.DMA,
        pltpu.SemaphoreType.DMA,
    ],
    compiler_params=pltpu.CompilerParams(use_tc_tiling_on_sc=False),
)
def _edges0(srcp, dstp, srcc, dstc, ap0, ap1, gbp0, gbp1, ac0, ac1,
            gbc0, gbc1, z0, out,
            sv3, dv3, dvS, av3, gbv3, mv3, svt, dvt, avt, gbvt, mvt, acc,
            g0s, g1s, g2s, s0s, s1s, s2s, i0s, i1s, i2s, tsem):
    gsem = (g0s, g1s, g2s)
    ssem = (s0s, s1s, s2s)
    isem = (i0s, i1s, i2s)
    c = lax.axis_index("c")
    s = lax.axis_index("s")
    wid = c * NS + s
    base = wid * EPT

    def zero_acc():
        @pl.when(s < NS - 1)
        def _():
            pltpu.sync_copy(z0.at[pl.ds(s * ROWS0, ROWS0)],
                            acc.at[pl.ds(s * ROWS0, ROWS0)])

        @pl.when(s == NS - 1)
        def _():
            pltpu.sync_copy(z0.at[pl.ds((NS - 1) * ROWS0, ROWS0_LAST)],
                            acc.at[pl.ds((NS - 1) * ROWS0, ROWS0_LAST)])

    def run_relation(src_h, dst_h, a_h, gb_h):
        def fire_gathers(b):
            for h in range(2):
                pltpu.async_copy(a_h.at[sv3.at[b, h]],
                                 av3.at[b, pl.ds(h * CH, CH)], gsem[b])
                pltpu.async_copy(gb_h.at[dv3.at[b, h]],
                                 gbv3.at[b, pl.ds(h * CH, CH)], gsem[b])

        def wait_gathers(b):
            for h in range(2):
                pltpu.make_async_copy(
                    a_h.at[sv3.at[b, h]],
                    av3.at[b, pl.ds(h * CH, CH)], gsem[b]).wait()
                pltpu.make_async_copy(
                    gb_h.at[dv3.at[b, h]],
                    gbv3.at[b, pl.ds(h * CH, CH)], gsem[b]).wait()

        def fire_scatters(b):
            for h in range(2):
                pltpu.async_copy(mv3.at[b, pl.ds(h * CH, CH)],
                                 acc.at[dvS.at[b, h]], ssem[b], add=True)

        def wait_scatters(b):
            for h in range(2):
                pltpu.make_async_copy(mv3.at[b, pl.ds(h * CH, CH)],
                                      acc.at[dvS.at[b, h]], ssem[b]).wait()

        for b in range(NB):
            off = base + b * CH2
            for h in range(2):
                pltpu.sync_copy(src_h.at[pl.ds(off + h * CH, CH)],
                                sv3.at[b, h])
                pltpu.sync_copy(dst_h.at[pl.ds(off + h * CH, CH)],
                                dv3.at[b, h])
            fire_gathers(b)

        def outer(g, carry):
            for b in range(NB):
                off_next = base + ((g + 1) * NB + b) * CH2
                wait_gathers(b)

                @pl.when(g > 0)
                def _():
                    wait_scatters(b)

                for t in range(16):
                    dvS[b, t // 8, pl.ds((t % 8) * 16, 16)] = (
                        dv3[b, t // 8, pl.ds((t % 8) * 16, 16)])

                @pl.when(g < NOUT - 1)
                def _():
                    for h in range(2):
                        pltpu.async_copy(
                            src_h.at[pl.ds(off_next + h * CH, CH)],
                            sv3.at[b, h], isem[b])
                        pltpu.async_copy(
                            dst_h.at[pl.ds(off_next + h * CH, CH)],
                            dv3.at[b, h], isem[b])

                def cbody(it, carry):
                    i0 = it * 8
                    for u in range(8):
                        i = i0 + u
                        a0 = av3[b, i, pl.ds(0, 16)]
                        gg = gbv3[b, i, pl.ds(0, 16)]
                        bb = gbv3[b, i, pl.ds(16, 16)]
                        mv3[b, i, pl.ds(0, 16)] = jnp.maximum(
                            gg * a0 + bb, 0.0)
                    return carry

                lax.fori_loop(0, CH2 // 8, cbody, 0)

                fire_scatters(b)

                @pl.when(g < NOUT - 1)
                def _():
                    for h in range(2):
                        pltpu.make_async_copy(
                            src_h.at[pl.ds(off_next + h * CH, CH)],
                            sv3.at[b, h], isem[b]).wait()
                        pltpu.make_async_copy(
                            dst_h.at[pl.ds(off_next + h * CH, CH)],
                            dv3.at[b, h], isem[b]).wait()
                    fire_gathers(b)

            return carry

        lax.fori_loop(0, NOUT, outer, 0)
        for b in range(NB):
            wait_scatters(b)

        off = base + NCH2 * CH2
        pltpu.sync_copy(src_h.at[pl.ds(off, TAIL)], svt)
        pltpu.sync_copy(dst_h.at[pl.ds(off, TAIL)], dvt)
        cp1 = pltpu.async_copy(a_h.at[svt], avt, tsem)
        cp2 = pltpu.async_copy(gb_h.at[dvt], gbvt, tsem)
        cp1.wait()
        cp2.wait()

        for i in range(TAIL):
            a0 = avt[i, pl.ds(0, 16)]
            gg = gbvt[i, pl.ds(0, 16)]
            bb = gbvt[i, pl.ds(16, 16)]
            mvt[i, pl.ds(0, 16)] = jnp.maximum(gg * a0 + bb, 0.0)
        pltpu.sync_copy(mvt, acc.at[dvt], add=True)

    for half, tabs in enumerate((((srcp, dstp, ap0, gbp0),
                                  (srcc, dstc, ac0, gbc0)),
                                 ((srcp, dstp, ap1, gbp1),
                                  (srcc, dstc, ac1, gbc1)))):
        zero_acc()
        plsc.subcore_barrier()
        for rel in tabs:
            run_relation(*rel)
        plsc.subcore_barrier()

        @pl.when(s < NS - 1)
        def _():
            pltpu.sync_copy(acc.at[pl.ds(s * ROWS0, ROWS0)],
                            out.at[c, half, pl.ds(s * ROWS0, ROWS0)])

        @pl.when(s == NS - 1)
        def _():
            pltpu.sync_copy(acc.at[pl.ds((NS - 1) * ROWS0, ROWS0_LAST)],
                            out.at[c, half,
                                   pl.ds((NS - 1) * ROWS0, ROWS0_LAST)])

        plsc.subcore_barrier()


def _prep1_body(s0, parts, wt, t_out, s1_out):
    p = parts[...]
    lo = p[0, 0] + p[1, 0]
    hi = p[0, 1] + p[1, 1]
    h1 = jnp.maximum(s0[...] + jnp.concatenate([lo, hi], axis=-1), 0.0)
    t = jnp.dot(h1, wt[...], preferred_element_type=jnp.float32)
    t_out[...] = t
    s1_out[...] = t[:, 0:1]


def _prep1(s0, parts, wt):
    return pl.pallas_call(
        _prep1_body,
        grid=(GRID,),
        in_specs=[
            pl.BlockSpec((RB, HID), lambda k: (k, 0)),
            pl.BlockSpec((NC, 2, RB, HH), lambda k: (0, 0, k, 0)),
            pl.BlockSpec((HID, 16), lambda k: (0, 0)),
        ],
        out_specs=[
            pl.BlockSpec((RB, 16), lambda k: (k, 0)),
            pl.BlockSpec((RB, 1), lambda k: (k, 0)),
        ],
        out_shape=[
            jax.ShapeDtypeStruct((N, 16), jnp.float32),
            jax.ShapeDtypeStruct((N, 1), jnp.float32),
        ],
    )(s0, parts, wt)


@functools.partial(
    pl.kernel,
    out_type=jax.ShapeDtypeStruct((NC, NP1), jnp.float32),
    mesh=_mesh,
    scratch_types=[
        pltpu.VMEM((NB, 2, CH), jnp.int32),
        pltpu.VMEM((NB, 2, CH), jnp.int32),
        pltpu.VMEM((NB, 2, CH), jnp.int32),
        pltpu.VMEM((NB * CH2,), jnp.float32),
        pltpu.VMEM((NB * CH2,), jnp.float32),
        pltpu.VMEM((NB * CH2,), jnp.float32),
        pltpu.VMEM((NB * CH2,), jnp.float32),
        pltpu.VMEM((TAIL,), jnp.int32),
        pltpu.VMEM((TAIL,), jnp.int32),
        pltpu.VMEM((TAIL,), jnp.float32),
        pltpu.VMEM((TAIL,), jnp.float32),
        pltpu.VMEM((TAIL,), jnp.float32),
        pltpu.VMEM((TAIL,), jnp.float32),
        pltpu.VMEM_SHARED((NP1,), jnp.float32),
        pltpu.SemaphoreType.DMA,
        pltpu.SemaphoreType.DMA,
        pltpu.SemaphoreType.DMA,
        pltpu.SemaphoreType.DMA,
        pltpu.SemaphoreType.DMA,
        pltpu.SemaphoreType.DMA,
        pltpu.SemaphoreType.DMA,
        pltpu.SemaphoreType.DMA,
        pltpu.SemaphoreType.DMA,
        pltpu.SemaphoreType.DMA,
    ],
    compiler_params=pltpu.CompilerParams(use_tc_tiling_on_sc=False),
)
def _edges1(srcp, dstp, srcc, dstc, a_p, g_p, b_p, a_c, g_c, b_c, z1, out,
            sv3, dv3, dvS, av3, gv3, bv3, mv3,
            svt, dvt, avt, gvt, bvt, mvt, acc,
            g0s, g1s, g2s, s0s, s1s, s2s, i0s, i1s, i2s, tsem):
    gsem = (g0s, g1s, g2s)
    ssem = (s0s, s1s, s2s)
    isem = (i0s, i1s, i2s)
    c = lax.axis_index("c")
    s = lax.axis_index("s")
    wid = c * NS + s
    base = wid * EPT
    pltpu.sync_copy(z1.at[pl.ds(s * RPT1, RPT1)],
                    acc.at[pl.ds(s * RPT1, RPT1)])
    plsc.subcore_barrier()

    def run_relation(src_h, dst_h, a_h, g_h, b_h):
        def fire_gathers(b):
            for h in range(2):
                pltpu.async_copy(a_h.at[sv3.at[b, h]],
                                 av3.at[pl.ds(b * CH2 + h * CH, CH)], gsem[b])
                pltpu.async_copy(g_h.at[dv3.at[b, h]],
                                 gv3.at[pl.ds(b * CH2 + h * CH, CH)], gsem[b])
                pltpu.async_copy(b_h.at[dv3.at[b, h]],
                                 bv3.at[pl.ds(b * CH2 + h * CH, CH)], gsem[b])

        def wait_gathers(b):
            for h in range(2):
                pltpu.make_async_copy(
                    a_h.at[sv3.at[b, h]],
                    av3.at[pl.ds(b * CH2 + h * CH, CH)], gsem[b]).wait()
                pltpu.make_async_copy(
                    g_h.at[dv3.at[b, h]],
                    gv3.at[pl.ds(b * CH2 + h * CH, CH)], gsem[b]).wait()
                pltpu.make_async_copy(
                    b_h.at[dv3.at[b, h]],
                    bv3.at[pl.ds(b * CH2 + h * CH, CH)], gsem[b]).wait()

        def fire_scatters(b):
            for h in range(2):
                pltpu.async_copy(mv3.at[pl.ds(b * CH2 + h * CH, CH)],
                                 acc.at[dvS.at[b, h]], ssem[b], add=True)

        def wait_scatters(b):
            for h in range(2):
                pltpu.make_async_copy(mv3.at[pl.ds(b * CH2 + h * CH, CH)],
                                      acc.at[dvS.at[b, h]], ssem[b]).wait()

        for b in range(NB):
            off = base + b * CH2
            for h in range(2):
                pltpu.sync_copy(src_h.at[pl.ds(off + h * CH, CH)],
                                sv3.at[b, h])
                pltpu.sync_copy(dst_h.at[pl.ds(off + h * CH, CH)],
                                dv3.at[b, h])
            fire_gathers(b)

        def outer(g, carry):
            for b in range(NB):
                off_next = base + ((g + 1) * NB + b) * CH2
                wait_gathers(b)

                @pl.when(g > 0)
                def _():
                    wait_scatters(b)

                for t in range(16):
                    dvS[b, t // 8, pl.ds((t % 8) * 16, 16)] = (
                        dv3[b, t // 8, pl.ds((t % 8) * 16, 16)])

                @pl.when(g < NOUT - 1)
                def _():
                    for h in range(2):
                        pltpu.async_copy(
                            src_h.at[pl.ds(off_next + h * CH, CH)],
                            sv3.at[b, h], isem[b])
                        pltpu.async_copy(
                            dst_h.at[pl.ds(off_next + h * CH, CH)],
                            dv3.at[b, h], isem[b])

                for j in range(CH2 // 16):
                    i0 = b * CH2 + j * 16
                    mv3[pl.ds(i0, 16)] = jnp.maximum(
                        gv3[pl.ds(i0, 16)] * av3[pl.ds(i0, 16)]
                        + bv3[pl.ds(i0, 16)], 0.0)

                fire_scatters(b)

                @pl.when(g < NOUT - 1)
                def _():
                    for h in range(2):
                        pltpu.make_async_copy(
                            src_h.at[pl.ds(off_next + h * CH, CH)],
                            sv3.at[b, h], isem[b]).wait()
                        pltpu.make_async_copy(
                            dst_h.at[pl.ds(off_next + h * CH, CH)],
                            dv3.at[b, h], isem[b]).wait()
                    fire_gathers(b)

            return carry

        lax.fori_loop(0, NOUT, outer, 0)
        for b in range(NB):
            wait_scatters(b)

        off = base + NCH2 * CH2
        pltpu.sync_copy(src_h.at[pl.ds(off, TAIL)], svt)
        pltpu.sync_copy(dst_h.at[pl.ds(off, TAIL)], dvt)
        cps = [pltpu.async_copy(a_h.at[svt], avt, tsem),
               pltpu.async_copy(g_h.at[dvt], gvt, tsem),
               pltpu.async_copy(b_h.at[dvt], bvt, tsem)]
        for cp in cps:
            cp.wait()
        for j in range(TAIL // 16):
            i0 = j * 16
            mvt[pl.ds(i0, 16)] = jnp.maximum(
                gvt[pl.ds(i0, 16)] * avt[pl.ds(i0, 16)]
                + bvt[pl.ds(i0, 16)], 0.0)
        pltpu.sync_copy(mvt, acc.at[dvt], add=True)

    run_relation(srcp, dstp, a_p, g_p, b_p)
    run_relation(srcc, dstc, a_c, g_c, b_c)

    plsc.subcore_barrier()
    pltpu.sync_copy(acc.at[pl.ds(s * RPT1, RPT1)],
                    out.at[c, pl.ds(s * RPT1, RPT1)])


def _mlp_body(s1r, p0r, p1r, w0, b0, w1, b1, w2, b2, w3, b3, w4, b4, w5, b5,
              out, acc):
    k = pl.program_id(0)

    @pl.when(k == 0)
    def _():
        acc[...] = jnp.zeros((1, 128), jnp.float32)

    v = (s1r[pl.ds(k, 1), :] + p0r[pl.ds(k, 1), :] + p1r[pl.ds(k, 1), :])
    acc[...] += jnp.dot(v, w0[0], preferred_element_type=jnp.float32)

    @pl.when(k == GRID - 1)
    def _():
        x = jnp.maximum(acc[...] + b0[...], 0.0)
        for w, b in ((w1, b1), (w2, b2), (w3, b3), (w4, b4)):
            x = jnp.maximum(
                jnp.dot(x, w[...], preferred_element_type=jnp.float32)
                + b[...], 0.0)
        out[...] = (jnp.dot(x, w5[...], preferred_element_type=jnp.float32)
                    + b5[...])


def _mlp(s1r, p0r, p1r, w0, b0, w1, b1, w2, b2, w3, b3, w4, b4, w5, b5):
    row = pl.BlockSpec((GRID, RB), lambda k: (0, 0))
    full = lambda shape: pl.BlockSpec(shape, lambda k: (0, 0))
    return pl.pallas_call(
        _mlp_body,
        grid=(GRID,),
        in_specs=[
            row, row, row,
            pl.BlockSpec((1, RB, 128), lambda k: (k, 0, 0)),
            full((1, 128)),
            full((128, 128)), full((1, 128)),
            full((128, 128)), full((1, 128)),
            full((128, 128)), full((1, 128)),
            full((128, 128)), full((1, 128)),
            full((128, 1)), full((1, 1)),
        ],
        out_specs=pl.BlockSpec((1, 1), lambda k: (0, 0)),
        out_shape=jax.ShapeDtypeStruct((1, 1), jnp.float32),
        scratch_shapes=[pltpu.VMEM((1, 128), jnp.float32)],
    )(s1r, p0r, p1r, w0, b0, w1, b1, w2, b2, w3, b3, w4, b4, w5, b5)


def kernel(node_features, edges_power, edges_comm, gnn_Wself_0,
           gnn_Wrel_0_power, gnn_Wfilm_0_power, gnn_Wrel_0_comm,
           gnn_Wfilm_0_comm, gnn_Wself_1, gnn_Wrel_1_power,
           gnn_Wfilm_1_power, gnn_Wrel_1_comm, gnn_Wfilm_1_comm,
           mlp_W0, mlp_b0, mlp_W1, mlp_b1, mlp_W2, mlp_b2, mlp_W3, mlp_b3,
           mlp_W4, mlp_b4, mlp_W5, mlp_b5):
    srcp, dstp = edges_power[0], edges_power[1]
    srcc, dstc = edges_comm[0], edges_comm[1]

    inv = (1.0 / jnp.array([4.0, 1.0, 2.0, 1.0, 230.0],
                           dtype=jnp.float32))[:, None]
    wrp = gnn_Wrel_0_power * inv
    wrc = gnn_Wrel_0_comm * inv
    wfp = gnn_Wfilm_0_power * inv
    wfc = gnn_Wfilm_0_comm * inv
    fp0 = jnp.concatenate([wfp[:, 0:HH], wfp[:, HID:HID + HH]], axis=1)
    fp1 = jnp.concatenate([wfp[:, HH:HID], wfp[:, HID + HH:]], axis=1)
    fc0 = jnp.concatenate([wfc[:, 0:HH], wfc[:, HID:HID + HH]], axis=1)
    fc1 = jnp.concatenate([wfc[:, HH:HID], wfc[:, HID + HH:]], axis=1)

    s0, ap0, ap1, gbp0, gbp1, ac0, ac1, gbc0, gbc1 = _prep0(
        node_features, gnn_Wself_0 * inv, wrp[:, :HH], wrp[:, HH:],
        fp0, fp1, wrc[:, :HH], wrc[:, HH:], fc0, fc1)

    z0 = jnp.zeros((N, HH), jnp.float32)
    parts0 = _edges0(srcp, dstp, srcc, dstc, ap0, ap1, gbp0, gbp1,
                     ac0, ac1, gbc0, gbc1, z0)

    wt = jnp.concatenate(
        [gnn_Wself_1, gnn_Wrel_1_power, gnn_Wrel_1_comm,
         gnn_Wfilm_1_power, gnn_Wfilm_1_comm,
         jnp.zeros((HID, 9), jnp.float32)], axis=1)
    tbl, s1 = _prep1(s0, parts0, wt)

    z1 = jnp.zeros((NP1,), jnp.float32)
    parts1 = _edges1(srcp, dstp, srcc, dstc,
                     tbl[:, 1], tbl[:, 3], tbl[:, 4],
                     tbl[:, 2], tbl[:, 5], tbl[:, 6], z1)

    s1r = s1.reshape(GRID, RB)
    p0r = parts1[0, :N].reshape(GRID, RB)
    p1r = parts1[1, :N].reshape(GRID, RB)
    res = _mlp(s1r, p0r, p1r, mlp_W0.reshape(GRID, RB, 128),
               mlp_b0.reshape(1, 128),
               mlp_W1, mlp_b1.reshape(1, 128), mlp_W2, mlp_b2.reshape(1, 128),
               mlp_W3, mlp_b3.reshape(1, 128), mlp_W4, mlp_b4.reshape(1, 128),
               mlp_W5, mlp_b5.reshape(1, 1))
    return res.reshape(1)
